# baseline (speedup 1.0000x reference)
import functools

import jax
import jax.numpy as jnp
from jax import lax
from jax.experimental import pallas as pl
from jax.experimental.pallas import tpu as pltpu
from jax.experimental.pallas import tpu_sc as plsc

D_VOCAB = 100000
N_VARS = 4
D_VAR = 16
D_MODEL = N_VARS * D_VAR

BATCH = 16384
HIST = 50

NC, NS = 2, 16
NW = NC * NS
VB = 4096
CHUNK = 256
N_BUF = 2

TB = 16384


def _softmax_t_block(w_ref, out_ref):
    X = w_ref[...]
    ys = []
    for g in range(N_VARS):
        sub = X[g * D_VAR:(g + 1) * D_VAR, :]
        m = jnp.max(sub, axis=0, keepdims=True)
        e = jnp.exp(sub - m)
        s = jnp.sum(e, axis=0, keepdims=True)
        ys.append(e / s)
    y = jnp.concatenate(ys, axis=0).T
    y3 = y.reshape(VB // 2, 2, D_MODEL)
    out_ref[...] = jnp.concatenate([y3[:, 0, :], y3[:, 1, :]], axis=1)


def _softmax_table(W_E):
    return pl.pallas_call(
        _softmax_t_block,
        grid=(pl.cdiv(D_VOCAB, VB),),
        in_specs=[pl.BlockSpec((D_MODEL, VB), lambda i: (0, i))],
        out_specs=pl.BlockSpec((VB // 2, 2 * D_MODEL), lambda i: (i, 0)),
        out_shape=jax.ShapeDtypeStruct((D_VOCAB // 2, 2 * D_MODEL),
                                       jnp.float32),
    )(W_E)


@functools.lru_cache(maxsize=None)
def _make_gather(n_rows):
    b_per_w = n_rows // NW
    n_chunks = b_per_w // CHUNK
    n_pairs = n_chunks // N_BUF
    assert n_chunks % N_BUF == 0 and n_chunks * CHUNK == b_per_w
    mesh = plsc.VectorSubcoreMesh(core_axis_name="c", subcore_axis_name="s")

    @functools.partial(
        pl.kernel, mesh=mesh,
        compiler_params=pltpu.CompilerParams(use_tc_tiling_on_sc=False),
        out_type=jax.ShapeDtypeStruct((n_rows // 2, 2 * D_MODEL),
                                      jnp.float32),
        scratch_types=[
            pltpu.VMEM((n_chunks, CHUNK), jnp.int32),
            pltpu.VMEM((N_BUF, CHUNK, D_MODEL), jnp.float32),
            pltpu.SemaphoreType.DMA,
            pltpu.SemaphoreType.DMA,
            pltpu.SemaphoreType.DMA,
            pltpu.SemaphoreType.DMA,
        ],
    )
    def gather(table_hbm, idx_hbm, out_hbm, idx_v, rows_v, g0, g1, o0, o1):
        wid = lax.axis_index("s") * NC + lax.axis_index("c")
        base = wid * b_per_w
        gsems = (g0, g1)
        osems = (o0, o1)

        pltpu.sync_copy(idx_hbm.at[wid], idx_v)

        def start_gather(c, b):
            pltpu.async_copy(table_hbm.at[idx_v.at[c]], rows_v.at[b], gsems[b])

        def out_copy(c, b):
            off = pl.multiple_of(base, CHUNK) + c * CHUNK
            h = off // BATCH
            r = off % BATCH
            half = r // (BATCH // 2)
            dstrow = h * (BATCH // 2) + r - half * (BATCH // 2)
            return pltpu.make_async_copy(
                rows_v.at[b],
                out_hbm.at[pl.ds(dstrow, CHUNK),
                           pl.ds(half * D_MODEL, D_MODEL)],
                osems[b])

        for b in range(N_BUF):
            start_gather(b, b)

        def pair(p, carry):
            for b in range(N_BUF):
                c = p * N_BUF + b
                pltpu.make_async_copy(table_hbm.at[idx_v.at[c]],
                                      rows_v.at[b], gsems[b]).wait()
                out_copy(c, b).start()
                nxt = c + N_BUF

                @pl.when(nxt < n_chunks)
                def _():
                    out_copy(c, b).wait()
                    start_gather(nxt, b)

            return carry

        lax.fori_loop(0, n_pairs, pair, 0)
        for b in range(N_BUF):
            out_copy(n_chunks - N_BUF + b, b).wait()

    return gather


def _transpose_block(o_ref, out_ref):
    t = o_ref[...].T
    out_ref[...] = jnp.concatenate([t[:D_MODEL], t[D_MODEL:]], axis=1)[None]


def _transpose_block_acc(o_ref, acc_ref, out_ref):
    del acc_ref
    _transpose_block(o_ref, out_ref)


N_SEG = 2


def _transpose_seg(O, seg, prev):
    h_seg = HIST // N_SEG
    h0 = seg * h_seg
    out_shape = jax.ShapeDtypeStruct((HIST, D_MODEL, BATCH), jnp.float32)
    in_spec = pl.BlockSpec((BATCH // 2, 2 * D_MODEL), lambda h: (h, 0))
    out_spec = pl.BlockSpec((1, D_MODEL, BATCH), lambda h: (h + h0, 0, 0))
    if prev is None:
        return pl.pallas_call(
            _transpose_block,
            grid=(h_seg,),
            in_specs=[in_spec],
            out_specs=out_spec,
            out_shape=out_shape,
        )(O)
    return pl.pallas_call(
        _transpose_block_acc,
        grid=(h_seg,),
        in_specs=[in_spec, pl.BlockSpec(memory_space=pl.ANY)],
        out_specs=out_spec,
        out_shape=out_shape,
        input_output_aliases={1: 0},
    )(O, prev)


def kernel(x, W_E):
    n = BATCH * HIST
    n_seg = n // N_SEG
    idx = x.T.reshape(N_SEG, NW, n_seg // NW // CHUNK, CHUNK).astype(
        jnp.int32)
    table = _softmax_table(W_E).reshape(D_VOCAB, D_MODEL)
    gather = _make_gather(n_seg)
    Pt = None
    for seg in range(N_SEG):
        O = gather(table, idx[seg])
        Pt = _transpose_seg(O, seg, Pt)
    return Pt.transpose(2, 0, 1)


# speedup vs baseline: 1.3326x; 1.3326x over previous
import functools

import jax
import jax.numpy as jnp
from jax import lax
from jax.experimental import pallas as pl
from jax.experimental.pallas import tpu as pltpu
from jax.experimental.pallas import tpu_sc as plsc

D_VOCAB = 100000
N_VARS = 4
D_VAR = 16
D_MODEL = N_VARS * D_VAR

BATCH = 16384
HIST = 50

NC, NS = 2, 16
NW = NC * NS
VB = 4096
CHUNK = 256
N_BUF = 2

TB = 16384


def _softmax_t_block(w_ref, out_ref):
    X = w_ref[...]
    ys = []
    for g in range(N_VARS):
        sub = X[g * D_VAR:(g + 1) * D_VAR, :]
        m = jnp.max(sub, axis=0, keepdims=True)
        e = jnp.exp(sub - m)
        s = jnp.sum(e, axis=0, keepdims=True)
        ys.append(e / s)
    y = jnp.concatenate(ys, axis=0).T
    lo = lax.bitcast_convert_type(
        y[:, :D_MODEL // 2].astype(jnp.bfloat16), jnp.uint16
    ).astype(jnp.uint32)
    hi = lax.bitcast_convert_type(
        y[:, D_MODEL // 2:].astype(jnp.bfloat16), jnp.uint16
    ).astype(jnp.uint32)
    w = lo | (hi << 16)
    w4 = w.reshape(VB // 4, 4, D_MODEL // 2)
    out_ref[...] = jnp.concatenate([w4[:, j, :] for j in range(4)], axis=1)


def _softmax_table(W_E):
    return pl.pallas_call(
        _softmax_t_block,
        grid=(pl.cdiv(D_VOCAB, VB),),
        in_specs=[pl.BlockSpec((D_MODEL, VB), lambda i: (0, i))],
        out_specs=pl.BlockSpec((VB // 4, 2 * D_MODEL), lambda i: (i, 0)),
        out_shape=jax.ShapeDtypeStruct((D_VOCAB // 4, 2 * D_MODEL),
                                       jnp.uint32),
    )(W_E)


@functools.lru_cache(maxsize=None)
def _make_gather(n_rows):
    b_per_w = n_rows // NW
    n_chunks = b_per_w // CHUNK
    n_pairs = n_chunks // N_BUF
    assert n_chunks % N_BUF == 0 and n_chunks * CHUNK == b_per_w
    mesh = plsc.VectorSubcoreMesh(core_axis_name="c", subcore_axis_name="s")

    @functools.partial(
        pl.kernel, mesh=mesh,
        compiler_params=pltpu.CompilerParams(use_tc_tiling_on_sc=False),
        out_type=jax.ShapeDtypeStruct((n_rows // 4, 2 * D_MODEL),
                                      jnp.uint32),
        scratch_types=[
            pltpu.VMEM((n_chunks, CHUNK), jnp.int32),
            pltpu.VMEM((N_BUF, CHUNK, D_MODEL // 2), jnp.uint32),
            pltpu.SemaphoreType.DMA,
            pltpu.SemaphoreType.DMA,
            pltpu.SemaphoreType.DMA,
            pltpu.SemaphoreType.DMA,
        ],
    )
    def gather(table_hbm, idx_hbm, out_hbm, idx_v, rows_v, g0, g1, o0, o1):
        wid = lax.axis_index("s") * NC + lax.axis_index("c")
        base = wid * b_per_w
        gsems = (g0, g1)
        osems = (o0, o1)

        pltpu.sync_copy(idx_hbm.at[wid], idx_v)

        def start_gather(c, b):
            pltpu.async_copy(table_hbm.at[idx_v.at[c]], rows_v.at[b], gsems[b])

        def out_copy(c, b):
            off = pl.multiple_of(base, CHUNK) + c * CHUNK
            h = off // BATCH
            r = off % BATCH
            q = r // (BATCH // 4)
            dstrow = h * (BATCH // 4) + r - q * (BATCH // 4)
            return pltpu.make_async_copy(
                rows_v.at[b],
                out_hbm.at[pl.ds(dstrow, CHUNK),
                           pl.ds(q * (D_MODEL // 2), D_MODEL // 2)],
                osems[b])

        for b in range(N_BUF):
            start_gather(b, b)

        def pair(p, carry):
            for b in range(N_BUF):
                c = p * N_BUF + b
                pltpu.make_async_copy(table_hbm.at[idx_v.at[c]],
                                      rows_v.at[b], gsems[b]).wait()
                out_copy(c, b).start()
                nxt = c + N_BUF

                @pl.when(nxt < n_chunks)
                def _():
                    out_copy(c, b).wait()
                    start_gather(nxt, b)

            return carry

        lax.fori_loop(0, n_pairs, pair, 0)
        for b in range(N_BUF):
            out_copy(n_chunks - N_BUF + b, b).wait()

    return gather


def _transpose_block(o_ref, out_ref):
    t = o_ref[...].T
    outs = []
    for q in range(4):
        Q = t[q * (D_MODEL // 2):(q + 1) * (D_MODEL // 2)]
        lo = lax.bitcast_convert_type(
            (Q & 0xFFFF).astype(jnp.uint16), jnp.bfloat16
        ).astype(jnp.float32)
        hi = lax.bitcast_convert_type(
            (Q >> 16).astype(jnp.uint16), jnp.bfloat16
        ).astype(jnp.float32)
        outs.append(jnp.concatenate([lo, hi], axis=0))
    out_ref[...] = jnp.concatenate(outs, axis=1)[None]


def _transpose_block_acc(o_ref, acc_ref, out_ref):
    del acc_ref
    _transpose_block(o_ref, out_ref)


N_SEG = 2


def _transpose_seg(O, seg, prev):
    h_seg = HIST // N_SEG
    h0 = seg * h_seg
    out_shape = jax.ShapeDtypeStruct((HIST, D_MODEL, BATCH), jnp.float32)
    in_spec = pl.BlockSpec((BATCH // 4, 2 * D_MODEL), lambda h: (h, 0))
    out_spec = pl.BlockSpec((1, D_MODEL, BATCH), lambda h: (h + h0, 0, 0))
    if prev is None:
        return pl.pallas_call(
            _transpose_block,
            grid=(h_seg,),
            in_specs=[in_spec],
            out_specs=out_spec,
            out_shape=out_shape,
        )(O)
    return pl.pallas_call(
        _transpose_block_acc,
        grid=(h_seg,),
        in_specs=[in_spec, pl.BlockSpec(memory_space=pl.ANY)],
        out_specs=out_spec,
        out_shape=out_shape,
        input_output_aliases={1: 0},
    )(O, prev)


def kernel(x, W_E):
    n = BATCH * HIST
    n_seg = n // N_SEG
    idx = x.T.reshape(N_SEG, NW, n_seg // NW // CHUNK, CHUNK).astype(
        jnp.int32)
    table = _softmax_table(W_E).reshape(D_VOCAB, D_MODEL // 2)
    gather = _make_gather(n_seg)
    Pt = None
    for seg in range(N_SEG):
        O = gather(table, idx[seg])
        Pt = _transpose_seg(O, seg, Pt)
    return Pt.transpose(2, 0, 1)


# speedup vs baseline: 1.3418x; 1.0069x over previous
import functools

import jax
import jax.numpy as jnp
from jax import lax
from jax.experimental import pallas as pl
from jax.experimental.pallas import tpu as pltpu
from jax.experimental.pallas import tpu_sc as plsc

D_VOCAB = 100000
N_VARS = 4
D_VAR = 16
D_MODEL = N_VARS * D_VAR

BATCH = 16384
HIST = 50

NC, NS = 2, 16
NW = NC * NS
VB = 4096
CHUNK = 256
N_BUF = 2

TB = 16384


def _softmax_t_block(w_ref, out_ref):
    X = w_ref[...]
    ys = []
    for g in range(N_VARS):
        sub = X[g * D_VAR:(g + 1) * D_VAR, :]
        m = jnp.max(sub, axis=0, keepdims=True)
        e = jnp.exp(sub - m)
        s = jnp.sum(e, axis=0, keepdims=True)
        ys.append(e / s)
    y = jnp.concatenate(ys, axis=0).T
    lo = lax.bitcast_convert_type(
        y[:, :D_MODEL // 2].astype(jnp.bfloat16), jnp.uint16
    ).astype(jnp.uint32)
    hi = lax.bitcast_convert_type(
        y[:, D_MODEL // 2:].astype(jnp.bfloat16), jnp.uint16
    ).astype(jnp.uint32)
    w = lo | (hi << 16)
    w4 = w.reshape(VB // 4, 4, D_MODEL // 2)
    out_ref[...] = jnp.concatenate([w4[:, j, :] for j in range(4)], axis=1)


def _softmax_table(W_E):
    return pl.pallas_call(
        _softmax_t_block,
        grid=(pl.cdiv(D_VOCAB, VB),),
        in_specs=[pl.BlockSpec((D_MODEL, VB), lambda i: (0, i))],
        out_specs=pl.BlockSpec((VB // 4, 2 * D_MODEL), lambda i: (i, 0)),
        out_shape=jax.ShapeDtypeStruct((D_VOCAB // 4, 2 * D_MODEL),
                                       jnp.uint32),
    )(W_E)


@functools.lru_cache(maxsize=None)
def _make_gather(n_rows):
    b_per_w = n_rows // NW
    n_chunks = b_per_w // CHUNK
    n_pairs = n_chunks // N_BUF
    assert n_chunks % N_BUF == 0 and n_chunks * CHUNK == b_per_w
    mesh = plsc.VectorSubcoreMesh(core_axis_name="c", subcore_axis_name="s")

    @functools.partial(
        pl.kernel, mesh=mesh,
        compiler_params=pltpu.CompilerParams(use_tc_tiling_on_sc=False),
        out_type=jax.ShapeDtypeStruct((n_rows // 4, 2 * D_MODEL),
                                      jnp.uint32),
        scratch_types=[
            pltpu.VMEM((n_chunks, CHUNK), jnp.int32),
            pltpu.VMEM((N_BUF, CHUNK, D_MODEL // 2), jnp.uint32),
            pltpu.SemaphoreType.DMA,
            pltpu.SemaphoreType.DMA,
            pltpu.SemaphoreType.DMA,
            pltpu.SemaphoreType.DMA,
        ],
    )
    def gather(table_hbm, idx_hbm, out_hbm, idx_v, rows_v, g0, g1, o0, o1):
        wid = lax.axis_index("s") * NC + lax.axis_index("c")
        base = wid * b_per_w
        gsems = (g0, g1)
        osems = (o0, o1)

        pltpu.sync_copy(idx_hbm.at[wid], idx_v)

        def start_gather(c, b):
            pltpu.async_copy(table_hbm.at[idx_v.at[c]], rows_v.at[b], gsems[b])

        def out_copy(c, b):
            off = pl.multiple_of(base, CHUNK) + c * CHUNK
            h = off // BATCH
            r = off % BATCH
            q = r // (BATCH // 4)
            dstrow = h * (BATCH // 4) + r - q * (BATCH // 4)
            return pltpu.make_async_copy(
                rows_v.at[b],
                out_hbm.at[pl.ds(dstrow, CHUNK),
                           pl.ds(q * (D_MODEL // 2), D_MODEL // 2)],
                osems[b])

        for b in range(N_BUF):
            start_gather(b, b)

        def pair(p, carry):
            for b in range(N_BUF):
                c = p * N_BUF + b
                pltpu.make_async_copy(table_hbm.at[idx_v.at[c]],
                                      rows_v.at[b], gsems[b]).wait()
                out_copy(c, b).start()
                nxt = c + N_BUF

                @pl.when(nxt < n_chunks)
                def _():
                    out_copy(c, b).wait()
                    start_gather(nxt, b)

            return carry

        lax.fori_loop(0, n_pairs, pair, 0)
        for b in range(N_BUF):
            out_copy(n_chunks - N_BUF + b, b).wait()

    return gather


def _transpose_block(o_ref, out_ref):
    t = o_ref[...].T
    outs = []
    for q in range(4):
        Q = t[q * (D_MODEL // 2):(q + 1) * (D_MODEL // 2)]
        lo = lax.bitcast_convert_type(
            (Q & 0xFFFF).astype(jnp.uint16), jnp.bfloat16
        ).astype(jnp.float32)
        hi = lax.bitcast_convert_type(
            (Q >> 16).astype(jnp.uint16), jnp.bfloat16
        ).astype(jnp.float32)
        outs.append(jnp.concatenate([lo, hi], axis=0))
    out_ref[...] = jnp.concatenate(outs, axis=1)[None]


def _transpose_block_acc(o_ref, acc_ref, out_ref):
    del acc_ref
    _transpose_block(o_ref, out_ref)


N_SEG = 5


def _transpose_seg(O, seg, prev):
    h_seg = HIST // N_SEG
    h0 = seg * h_seg
    out_shape = jax.ShapeDtypeStruct((HIST, D_MODEL, BATCH), jnp.float32)
    in_spec = pl.BlockSpec((BATCH // 4, 2 * D_MODEL), lambda h: (h, 0))
    out_spec = pl.BlockSpec((1, D_MODEL, BATCH), lambda h: (h + h0, 0, 0))
    if prev is None:
        return pl.pallas_call(
            _transpose_block,
            grid=(h_seg,),
            in_specs=[in_spec],
            out_specs=out_spec,
            out_shape=out_shape,
        )(O)
    return pl.pallas_call(
        _transpose_block_acc,
        grid=(h_seg,),
        in_specs=[in_spec, pl.BlockSpec(memory_space=pl.ANY)],
        out_specs=out_spec,
        out_shape=out_shape,
        input_output_aliases={1: 0},
    )(O, prev)


def kernel(x, W_E):
    n = BATCH * HIST
    n_seg = n // N_SEG
    idx = x.T.reshape(N_SEG, NW, n_seg // NW // CHUNK, CHUNK).astype(
        jnp.int32)
    table = _softmax_table(W_E).reshape(D_VOCAB, D_MODEL // 2)
    gather = _make_gather(n_seg)
    Pt = None
    for seg in range(N_SEG):
        O = gather(table, idx[seg])
        Pt = _transpose_seg(O, seg, Pt)
    return Pt.transpose(2, 0, 1)


# speedup vs baseline: 1.3956x; 1.0401x over previous
import functools

import jax
import jax.numpy as jnp
from jax import lax
from jax.experimental import pallas as pl
from jax.experimental.pallas import tpu as pltpu
from jax.experimental.pallas import tpu_sc as plsc

D_VOCAB = 100000
N_VARS = 4
D_VAR = 16
D_MODEL = N_VARS * D_VAR

BATCH = 16384
HIST = 50

NC, NS = 2, 16
NW = NC * NS
VB = 4096
CHUNK = 256
N_BUF = 2

TB = 16384


def _softmax_t_block(w_ref, out_ref):
    X = w_ref[...]
    ys = []
    for g in range(N_VARS):
        sub = X[g * D_VAR:(g + 1) * D_VAR, :]
        m = jnp.max(sub, axis=0, keepdims=True)
        e = jnp.exp(sub - m)
        s = jnp.sum(e, axis=0, keepdims=True)
        ys.append(e / s)
    top = jnp.concatenate(ys[:2], axis=0)
    bot = jnp.concatenate(ys[2:], axis=0)
    lo = lax.bitcast_convert_type(
        top.astype(jnp.bfloat16), jnp.uint16).astype(jnp.uint32)
    hi = lax.bitcast_convert_type(
        bot.astype(jnp.bfloat16), jnp.uint16).astype(jnp.uint32)
    w = (lo | (hi << 16)).T
    w4 = w.reshape(VB // 4, 4, D_MODEL // 2)
    out_ref[...] = jnp.concatenate([w4[:, j, :] for j in range(4)], axis=1)


def _softmax_table(W_E):
    return pl.pallas_call(
        _softmax_t_block,
        grid=(pl.cdiv(D_VOCAB, VB),),
        in_specs=[pl.BlockSpec((D_MODEL, VB), lambda i: (0, i))],
        out_specs=pl.BlockSpec((VB // 4, 2 * D_MODEL), lambda i: (i, 0)),
        out_shape=jax.ShapeDtypeStruct((D_VOCAB // 4, 2 * D_MODEL),
                                       jnp.uint32),
    )(W_E)


@functools.lru_cache(maxsize=None)
def _make_gather(n_rows):
    b_per_w = n_rows // NW
    n_chunks = b_per_w // CHUNK
    n_pairs = n_chunks // N_BUF
    assert n_chunks % N_BUF == 0 and n_chunks * CHUNK == b_per_w
    mesh = plsc.VectorSubcoreMesh(core_axis_name="c", subcore_axis_name="s")

    @functools.partial(
        pl.kernel, mesh=mesh,
        compiler_params=pltpu.CompilerParams(use_tc_tiling_on_sc=False),
        out_type=jax.ShapeDtypeStruct((n_rows // 4, 2 * D_MODEL),
                                      jnp.uint32),
        scratch_types=[
            pltpu.VMEM((n_chunks, CHUNK), jnp.int32),
            pltpu.VMEM((N_BUF, CHUNK, D_MODEL // 2), jnp.uint32),
            pltpu.SemaphoreType.DMA,
            pltpu.SemaphoreType.DMA,
            pltpu.SemaphoreType.DMA,
            pltpu.SemaphoreType.DMA,
        ],
    )
    def gather(table_hbm, idx_hbm, out_hbm, idx_v, rows_v, g0, g1, o0, o1):
        wid = lax.axis_index("s") * NC + lax.axis_index("c")
        base = wid * b_per_w
        gsems = (g0, g1)
        osems = (o0, o1)

        pltpu.sync_copy(idx_hbm.at[wid], idx_v)

        def start_gather(c, b):
            pltpu.async_copy(table_hbm.at[idx_v.at[c]], rows_v.at[b], gsems[b])

        def out_copy(c, b):
            off = pl.multiple_of(base, CHUNK) + c * CHUNK
            h = off // BATCH
            r = off % BATCH
            q = r // (BATCH // 4)
            dstrow = h * (BATCH // 4) + r - q * (BATCH // 4)
            return pltpu.make_async_copy(
                rows_v.at[b],
                out_hbm.at[pl.ds(dstrow, CHUNK),
                           pl.ds(q * (D_MODEL // 2), D_MODEL // 2)],
                osems[b])

        for b in range(N_BUF):
            start_gather(b, b)

        def pair(p, carry):
            for b in range(N_BUF):
                c = p * N_BUF + b
                pltpu.make_async_copy(table_hbm.at[idx_v.at[c]],
                                      rows_v.at[b], gsems[b]).wait()
                out_copy(c, b).start()
                nxt = c + N_BUF

                @pl.when(nxt < n_chunks)
                def _():
                    out_copy(c, b).wait()
                    start_gather(nxt, b)

            return carry

        lax.fori_loop(0, n_pairs, pair, 0)
        for b in range(N_BUF):
            out_copy(n_chunks - N_BUF + b, b).wait()

    return gather


def _transpose_block(o_ref, out_ref):
    t = o_ref[...].T
    outs = []
    for q in range(4):
        Q = t[q * (D_MODEL // 2):(q + 1) * (D_MODEL // 2)]
        lo = lax.bitcast_convert_type(
            (Q & 0xFFFF).astype(jnp.uint16), jnp.bfloat16
        ).astype(jnp.float32)
        hi = lax.bitcast_convert_type(
            (Q >> 16).astype(jnp.uint16), jnp.bfloat16
        ).astype(jnp.float32)
        outs.append(jnp.concatenate([lo, hi], axis=0))
    out_ref[...] = jnp.concatenate(outs, axis=1)[None]


def _transpose_block_acc(o_ref, acc_ref, out_ref):
    del acc_ref
    _transpose_block(o_ref, out_ref)


N_SEG = 5


def _transpose_seg(O, seg, prev):
    h_seg = HIST // N_SEG
    h0 = seg * h_seg
    out_shape = jax.ShapeDtypeStruct((HIST, D_MODEL, BATCH), jnp.float32)
    in_spec = pl.BlockSpec((BATCH // 4, 2 * D_MODEL), lambda h: (h, 0))
    out_spec = pl.BlockSpec((1, D_MODEL, BATCH), lambda h: (h + h0, 0, 0))
    if prev is None:
        return pl.pallas_call(
            _transpose_block,
            grid=(h_seg,),
            in_specs=[in_spec],
            out_specs=out_spec,
            out_shape=out_shape,
        )(O)
    return pl.pallas_call(
        _transpose_block_acc,
        grid=(h_seg,),
        in_specs=[in_spec, pl.BlockSpec(memory_space=pl.ANY)],
        out_specs=out_spec,
        out_shape=out_shape,
        input_output_aliases={1: 0},
    )(O, prev)


def kernel(x, W_E):
    n = BATCH * HIST
    n_seg = n // N_SEG
    idx = x.T.reshape(N_SEG, NW, n_seg // NW // CHUNK, CHUNK).astype(
        jnp.int32)
    table = _softmax_table(W_E).reshape(D_VOCAB, D_MODEL // 2)
    gather = _make_gather(n_seg)
    Pt = None
    for seg in range(N_SEG):
        O = gather(table, idx[seg])
        Pt = _transpose_seg(O, seg, Pt)
    return Pt.transpose(2, 0, 1)
